# baseline (speedup 1.0000x reference)
import functools

import jax
import jax.numpy as jnp
import numpy as np
from jax.experimental import pallas as pl
from jax.experimental.pallas import tpu as pltpu

_N_EDGE = 1600000
_D_E = 16
_N_NODE = 50000
_D_N = 128
_FOLD = 128 // _D_E
_EDGE_ROWS = _N_EDGE // _FOLD
_GRID = 50
_EBLK = _EDGE_ROWS // _GRID
_NBLK = _N_NODE // _GRID


def _body(g_ref, e_ref, n_ref, w_ref, b_ref, o_ref, acc_e, acc_n):
    step = pl.program_id(0)

    @pl.when(step == 0)
    def _init():
        acc_e[...] = jnp.zeros_like(acc_e)
        acc_n[...] = jnp.zeros_like(acc_n)

    acc_e[...] += jnp.sum(e_ref[...], axis=0, keepdims=True)
    acc_n[...] += jnp.sum(n_ref[...], axis=0, keepdims=True)

    @pl.when(step == pl.num_programs(0) - 1)
    def _finish():
        v = jnp.concatenate(
            [
                g_ref[...],
                acc_e[...] * (1.0 / _N_EDGE),
                acc_n[...] * (1.0 / _N_NODE),
            ],
            axis=1,
        )
        o_ref[...] = (
            jax.lax.dot_general(
                v, w_ref[...], (((1,), (0,)), ((), ())),
                preferred_element_type=jnp.float32,
            )
            + b_ref[...]
        )


@jax.jit
def kernel(global_attr, edge_attr, node_attr, W, b):
    e_view = edge_attr.reshape(_EDGE_ROWS, 128)
    g_row = global_attr.reshape(1, 128)
    b_row = b.reshape(1, 128)
    w_e_exp = W[128 + (jnp.arange(128) % _D_E)]
    w_cat = jnp.concatenate([W[:128], w_e_exp, W[144:]], axis=0)

    out_row = pl.pallas_call(
        _body,
        grid=(_GRID,),
        in_specs=[
            pl.BlockSpec((1, 128), lambda i: (0, 0)),
            pl.BlockSpec((_EBLK, 128), lambda i: (i, 0)),
            pl.BlockSpec((_NBLK, 128), lambda i: (i, 0)),
            pl.BlockSpec((384, 128), lambda i: (0, 0)),
            pl.BlockSpec((1, 128), lambda i: (0, 0)),
        ],
        out_specs=pl.BlockSpec((1, 128), lambda i: (0, 0)),
        out_shape=jax.ShapeDtypeStruct((1, 128), jnp.float32),
        scratch_shapes=[
            pltpu.VMEM((1, 128), jnp.float32),
            pltpu.VMEM((1, 128), jnp.float32),
        ],
    )(g_row, e_view, node_attr, w_cat, b_row)
    return out_row.reshape(128)


# speedup vs baseline: 11.7970x; 11.7970x over previous
import jax
import jax.numpy as jnp
import numpy as np
from jax.experimental import pallas as pl
from jax.experimental.pallas import tpu as pltpu

_N_EDGE = 1600000
_D_E = 16
_N_NODE = 50000
_GRID = 50
_CW = _N_EDGE // _GRID
_NBLK = _N_NODE // _GRID


def _body(g_ref, e_ref, n_ref, w_ref, b_ref, o_ref, acc_e, acc_n):
    step = pl.program_id(0)

    @pl.when(step == 0)
    def _init():
        acc_e[...] = jnp.zeros_like(acc_e)
        acc_n[...] = jnp.zeros_like(acc_n)

    acc_e[...] += e_ref[...]
    acc_n[...] += jnp.sum(n_ref[...], axis=0, keepdims=True)

    @pl.when(step == pl.num_programs(0) - 1)
    def _finish():
        e_sum = jnp.sum(acc_e[...], axis=1, keepdims=True)
        dn = (((1,), (0,)), ((), ()))
        out = jax.lax.dot_general(
            g_ref[...], w_ref[0:128, :], dn,
            preferred_element_type=jnp.float32,
        )
        out += jax.lax.dot_general(
            e_sum * (1.0 / _N_EDGE), w_ref[128:144, :],
            (((0,), (0,)), ((), ())),
            preferred_element_type=jnp.float32,
        )
        out += jax.lax.dot_general(
            acc_n[...] * (1.0 / _N_NODE), w_ref[144:272, :], dn,
            preferred_element_type=jnp.float32,
        )
        o_ref[...] = out + b_ref[...]


@jax.jit
def kernel(global_attr, edge_attr, node_attr, W, b):
    e_t = edge_attr.T
    g_row = global_attr.reshape(1, 128)
    b_row = b.reshape(1, 128)

    out_row = pl.pallas_call(
        _body,
        grid=(_GRID,),
        in_specs=[
            pl.BlockSpec((1, 128), lambda i: (0, 0)),
            pl.BlockSpec((_D_E, _CW), lambda i: (0, i)),
            pl.BlockSpec((_NBLK, 128), lambda i: (i, 0)),
            pl.BlockSpec((272, 128), lambda i: (0, 0)),
            pl.BlockSpec((1, 128), lambda i: (0, 0)),
        ],
        out_specs=pl.BlockSpec((1, 128), lambda i: (0, 0)),
        out_shape=jax.ShapeDtypeStruct((1, 128), jnp.float32),
        scratch_shapes=[
            pltpu.VMEM((_D_E, _CW), jnp.float32),
            pltpu.VMEM((1, 128), jnp.float32),
        ],
    )(g_row, e_t, node_attr, W, b_row)
    return out_row.reshape(128)


# speedup vs baseline: 14.7590x; 1.2511x over previous
import jax
import jax.numpy as jnp
import numpy as np
from jax.experimental import pallas as pl
from jax.experimental.pallas import tpu as pltpu

_N_EDGE = 1600000
_D_E = 16
_N_NODE = 50000
_GRID = 25
_CW = _N_EDGE // _GRID
_NBLK = _N_NODE // _GRID


def _body(g_ref, e_ref, n_ref, w_ref, b_ref, o_ref, acc_e, acc_n):
    step = pl.program_id(0)

    @pl.when(step == 0)
    def _init():
        acc_e[...] = jnp.zeros_like(acc_e)
        acc_n[...] = jnp.zeros_like(acc_n)

    acc_e[...] += e_ref[...]
    acc_n[...] += jnp.sum(n_ref[...].reshape(_NBLK // 8, 8, 128), axis=0)

    @pl.when(step == pl.num_programs(0) - 1)
    def _finish():
        e_sum = jnp.sum(acc_e[...], axis=1, keepdims=True)
        dn = (((1,), (0,)), ((), ()))
        out = jax.lax.dot_general(
            g_ref[...], w_ref[0:128, :], dn,
            preferred_element_type=jnp.float32,
        )
        out += jax.lax.dot_general(
            e_sum * (1.0 / _N_EDGE), w_ref[128:144, :],
            (((0,), (0,)), ((), ())),
            preferred_element_type=jnp.float32,
        )
        n_sum = jnp.sum(acc_n[...], axis=0, keepdims=True)
        out += jax.lax.dot_general(
            n_sum * (1.0 / _N_NODE), w_ref[144:272, :], dn,
            preferred_element_type=jnp.float32,
        )
        o_ref[...] = out + b_ref[...]


@jax.jit
def kernel(global_attr, edge_attr, node_attr, W, b):
    e_t = edge_attr.T
    g_row = global_attr.reshape(1, 128)
    b_row = b.reshape(1, 128)

    out_row = pl.pallas_call(
        _body,
        grid=(_GRID,),
        in_specs=[
            pl.BlockSpec((1, 128), lambda i: (0, 0)),
            pl.BlockSpec((_D_E, _CW), lambda i: (0, i)),
            pl.BlockSpec((_NBLK, 128), lambda i: (i, 0)),
            pl.BlockSpec((272, 128), lambda i: (0, 0)),
            pl.BlockSpec((1, 128), lambda i: (0, 0)),
        ],
        out_specs=pl.BlockSpec((1, 128), lambda i: (0, 0)),
        out_shape=jax.ShapeDtypeStruct((1, 128), jnp.float32),
        scratch_shapes=[
            pltpu.VMEM((_D_E, _CW), jnp.float32),
            pltpu.VMEM((8, 128), jnp.float32),
        ],
    )(g_row, e_t, node_attr, W, b_row)
    return out_row.reshape(128)
